# baseline (speedup 1.0000x reference)
import jax
import jax.numpy as jnp
from jax import lax
from jax.experimental import pallas as pl
from jax.experimental.pallas import tpu as pltpu
from jax.experimental.pallas import tpu_sc as plsc

B = 16384
D = 64
L = 16
NC = 2
NS = 16
NW = NC * NS
ROWS_W = B // NW
CHUNK = 128
NCH = ROWS_W // CHUNK
NDR = D // L
NENT = 100000
NT = (NENT + 127) // 128
NR = NT * 128
HALF = NR // 2
BT = B // 128
OUT1D = B * D


def _lane_sum(x):
    i = lax.iota(jnp.int32, L)
    dnums = lax.GatherDimensionNumbers(
        offset_dims=(), collapsed_slice_dims=(0,), start_index_map=(0,))
    for k in (8, 4, 2, 1):
        x = x + lax.gather(x, (i ^ k)[:, None], dnums, slice_sizes=(1,),
                           mode=lax.GatherScatterMode.PROMISE_IN_BOUNDS)
    return x


def _vrsqrt(x):
    i = lax.bitcast_convert_type(x, jnp.int32)
    y = lax.bitcast_convert_type(jnp.int32(0x5F3759DF) - (i >> 1), jnp.float32)
    for _ in range(3):
        y = y * (1.5 - 0.5 * x * y * y)
    return y


def _pack_body(a0, b0, a1, b1, a2, b2, o0, o1, o2):
    for a, b, o in ((a0, b0, o0), (a1, b1, o1), (a2, b2, o2)):
        o[:, 0:D] = a[...].T
        o[:, D:2 * D] = b[...].T


def _pack_tables(a_t, f_t, d_t):
    nblk = HALF // CHUNK
    in_a = pl.BlockSpec((D, CHUNK), lambda t: (0, t))
    in_b = pl.BlockSpec((D, CHUNK), lambda t: (0, nblk + t))
    out_s = pl.BlockSpec((CHUNK, 2 * D), lambda t: (t, 0))
    return pl.pallas_call(
        _pack_body,
        grid=(nblk,),
        in_specs=[in_a, in_b, in_a, in_b, in_a, in_b],
        out_specs=[out_s, out_s, out_s],
        out_shape=tuple(jax.ShapeDtypeStruct((HALF, 2 * D), jnp.float32)
                        for _ in range(3)),
    )(a_t, a_t, f_t, f_t, d_t, d_t)


def _gather_body(uid, wr, ci, co, af, author, affil, doc, rel, hyper,
                 o_user, o_wr, o_ci, o_co, o_af, r_wr, r_ci, r_co, r_af,
                 idx_v, idx2_v, hoff_v, rows_v, tr_v, hyp_v, rel_v,
                 gsem, wsem):
    wid = lax.axis_index("s") * NC + lax.axis_index("c")
    base = wid * ROWS_W
    pltpu.sync_copy(hyper, hyp_v)
    pltpu.sync_copy(rel, rel_v)

    dof = []
    for dg in range(NDR):
        d = lax.iota(jnp.int32, L) + dg * L
        dof.append(((d >> 3) << 12) + ((d & 7) << 7))

    idx_refs = [uid, wr, ci, co, af]
    tabs = [author, doc, doc, author, affil]
    outs = [o_user, o_wr, o_ci, o_co, o_af]

    for r in range(5):
        pltpu.sync_copy(idx_refs[r].at[wid], idx_v)
        for j in range(NCH):
            def xbody(g, _, _j=j):
                v = idx_v[_j, pl.ds(g * L, L)]
                idx2_v[_j, pl.ds(g * L, L)] = jnp.where(v >= HALF, v - HALF, v)
                hoff_v[pl.ds(_j * CHUNK + g * L, L)] = (
                    jnp.where(v >= HALF, D, 0))
                return 0
            lax.fori_loop(0, CHUNK // L, xbody, 0)
        gc = [pltpu.async_copy(tabs[r].at[idx2_v.at[j]],
                               rows_v.at[pl.ds(j * CHUNK, CHUNK)], gsem)
              for j in range(NCH)]
        if r > 0:
            h = [hyp_v[r - 1, pl.ds(d * L, L)] for d in range(NDR)]
            nsq = jnp.maximum(
                _lane_sum(h[0] * h[0] + h[1] * h[1] + h[2] * h[2] + h[3] * h[3]),
                1e-24)
            inv = _vrsqrt(nsq)
            hn = [h[d] * inv for d in range(NDR)]
        for j in range(NCH):
            gc[j].wait()

            def body(i, _, _j=j, _r=r):
                rb = _j * CHUNK + i
                boff = _j * 1024 + i
                ho = hoff_v[pl.ds(rb, L)][0]
                e = [rows_v[rb, pl.ds(ho + d * L, L)] for d in range(NDR)]
                if _r > 0:
                    p = (e[0] * hn[0] + e[1] * hn[1]
                         + e[2] * hn[2] + e[3] * hn[3])
                    s = _lane_sum(p)
                    e = [e[d] - s * hn[d] for d in range(NDR)]
                for d in range(NDR):
                    plsc.store_scatter(tr_v, [dof[d] + boff], e[d])
                return 0
            lax.fori_loop(0, CHUNK, body, 0)
        wc = [pltpu.async_copy(
            tr_v.at[pl.ds(td * 4096, 4096)],
            outs[r].at[pl.ds(td * (BT * 1024) + base * 8, 4096)], wsem)
            for td in range(8)]
        for c in wc:
            c.wait()

    rel_outs = [r_wr, r_ci, r_co, r_af]
    for r in range(4):
        g = [rel_v[r, pl.ds(d * L, L)] for d in range(NDR)]

        def fbody(i, _, _g=g):
            for d in range(NDR):
                plsc.store_scatter(tr_v, [dof[d] + i], _g[d])
            return 0
        lax.fori_loop(0, CHUNK, fbody, 0)
        wc = []
        for td in range(8):
            for j in range(NCH):
                wc.append(pltpu.async_copy(
                    tr_v.at[pl.ds(td * 4096, 1024)],
                    rel_outs[r].at[pl.ds(td * (BT * 1024) + (base * 8)
                                         + j * 1024, 1024)], wsem))
        for c in wc:
            c.wait()


def kernel(user_id, wrote, cited, coauthor, affiliation,
           author_table, affil_table, doc_table, rel_table, hyper_table):
    def prep(x):
        return x.astype(jnp.int32).reshape(NW, NCH, CHUNK)

    a_pk, f_pk, d_pk = _pack_tables(author_table.T, affil_table.T,
                                    doc_table.T)

    mesh = plsc.VectorSubcoreMesh(core_axis_name="c", subcore_axis_name="s")
    g_f = pl.kernel(
        _gather_body,
        mesh=mesh,
        out_type=tuple(jax.ShapeDtypeStruct((OUT1D,), jnp.float32)
                       for _ in range(9)),
        compiler_params=pltpu.CompilerParams(use_tc_tiling_on_sc=False,
                                             needs_layout_passes=False),
        scratch_types=[
            pltpu.VMEM((NCH, CHUNK), jnp.int32),
            pltpu.VMEM((NCH, CHUNK), jnp.int32),
            pltpu.VMEM((ROWS_W + L,), jnp.int32),
            pltpu.VMEM((ROWS_W, 2 * D), jnp.float32),
            pltpu.VMEM((8 * NCH * 1024,), jnp.float32),
            pltpu.VMEM((4, D), jnp.float32),
            pltpu.VMEM((4, D), jnp.float32),
            pltpu.SemaphoreType.DMA,
            pltpu.SemaphoreType.DMA,
        ],
    )
    outs1d = g_f(prep(user_id), prep(wrote), prep(cited), prep(coauthor),
                 prep(affiliation), a_pk, f_pk, d_pk,
                 rel_table, hyper_table)

    def unbit(o):
        return (o.reshape(8, BT, 8, 128).transpose(1, 3, 0, 2)
                .reshape(B, D))

    return tuple(unbit(o) for o in outs1d)


# speedup vs baseline: 1.0991x; 1.0991x over previous
import jax
import jax.numpy as jnp
from jax import lax
from jax.experimental import pallas as pl
from jax.experimental.pallas import tpu as pltpu
from jax.experimental.pallas import tpu_sc as plsc

B = 16384
D = 64
L = 16
NC = 2
NS = 16
NW = NC * NS
ROWS_W = B // NW
CHUNK = 128
NCH = ROWS_W // CHUNK
NG = ROWS_W // L
NDR = D // L
PBLK = 512
HALF = 98 * PBLK
BT = B // 128
OUT1D = B * D


def _lane_sum(x):
    i = lax.iota(jnp.int32, L)
    dnums = lax.GatherDimensionNumbers(
        offset_dims=(), collapsed_slice_dims=(0,), start_index_map=(0,))
    for k in (8, 4, 2, 1):
        x = x + lax.gather(x, (i ^ k)[:, None], dnums, slice_sizes=(1,),
                           mode=lax.GatherScatterMode.PROMISE_IN_BOUNDS)
    return x


def _vrsqrt(x):
    i = lax.bitcast_convert_type(x, jnp.int32)
    y = lax.bitcast_convert_type(jnp.int32(0x5F3759DF) - (i >> 1), jnp.float32)
    for _ in range(3):
        y = y * (1.5 - 0.5 * x * y * y)
    return y


def _pack_body(a0, b0, a1, b1, a2, b2, o0, o1, o2):
    ri = lax.broadcasted_iota(jnp.int32, (D, D), 0)
    ci = lax.broadcasted_iota(jnp.int32, (D, D), 1)
    ident = jnp.where(ri == ci, 1.0, 0.0).astype(jnp.float32)
    dn = (((0,), (0,)), ((), ()))
    for a, b, o in ((a0, b0, o0), (a1, b1, o1), (a2, b2, o2)):
        o[:, 0:D] = lax.dot_general(a[...], ident, dn,
                                    precision=lax.Precision.HIGHEST,
                                    preferred_element_type=jnp.float32)
        o[:, D:2 * D] = lax.dot_general(b[...], ident, dn,
                                        precision=lax.Precision.HIGHEST,
                                        preferred_element_type=jnp.float32)


def _pack_tables(a_t, f_t, d_t):
    nblk = HALF // PBLK
    in_a = pl.BlockSpec((D, PBLK), lambda t: (0, t))
    in_b = pl.BlockSpec((D, PBLK), lambda t: (0, nblk + t))
    out_s = pl.BlockSpec((PBLK, 2 * D), lambda t: (t, 0))
    return pl.pallas_call(
        _pack_body,
        grid=(nblk,),
        in_specs=[in_a, in_b, in_a, in_b, in_a, in_b],
        out_specs=[out_s, out_s, out_s],
        out_shape=tuple(jax.ShapeDtypeStruct((HALF, 2 * D), jnp.float32)
                        for _ in range(3)),
    )(a_t, a_t, f_t, f_t, d_t, d_t)


def _gather_body(uid, wr, ci, co, af, author, affil, doc, rel, hyper,
                 o_user, o_wr, o_ci, o_co, o_af, r_wr, r_ci, r_co, r_af,
                 idx_v, idx2_v, hoff_v, rows_v, tr_v, hn_v, hyp_v, rel_v,
                 gsem, wsem):
    wid = lax.axis_index("s") * NC + lax.axis_index("c")
    base = wid * ROWS_W
    pltpu.sync_copy(hyper, hyp_v)
    pltpu.sync_copy(rel, rel_v)
    lane = lax.iota(jnp.int32, L)

    idx_refs = [uid, wr, ci, co, af]
    tabs = [author, doc, doc, author, affil]
    outs = [o_user, o_wr, o_ci, o_co, o_af]

    def dslot(d, gslot):
        return ((d >> 3) << 12) + ((d & 7) << 7) + gslot

    for r in range(5):
        pltpu.sync_copy(idx_refs[r].at[wid], idx_v)
        for j in range(NCH):
            def xbody(g, _, _j=j):
                v = idx_v[_j, pl.ds(g * L, L)]
                idx2_v[_j, pl.ds(g * L, L)] = jnp.where(v >= HALF, v - HALF, v)
                hoff_v[pl.ds(_j * CHUNK + g * L, L)] = (
                    jnp.where(v >= HALF, D, 0))
                return 0
            lax.fori_loop(0, CHUNK // L, xbody, 0)
        gc = [pltpu.async_copy(tabs[r].at[idx2_v.at[j]],
                               rows_v.at[pl.ds(j * CHUNK, CHUNK)], gsem)
              for j in range(NCH)]
        if r > 0:
            h = [hyp_v[r - 1, pl.ds(dg * L, L)] for dg in range(NDR)]
            nsq = jnp.maximum(
                _lane_sum(h[0] * h[0] + h[1] * h[1] + h[2] * h[2] + h[3] * h[3]),
                1e-24)
            inv = _vrsqrt(nsq)
            for dg in range(NDR):
                hn_v[pl.ds(dg * L, L)] = h[dg] * inv
        for j in range(NCH):
            gc[j].wait()

            def body(gg, _, _j=j, _r=r):
                g = _j * (CHUNK // L) + gg
                rows16 = g * L + lane
                cols = hoff_v[pl.ds(g * L, L)]
                gslot = ((g >> 3) << 10) + ((g & 7) << 4)
                if _r > 0:
                    def p1(d, acc):
                        c = plsc.load_gather(rows_v, [rows16, cols + d])
                        hv = plsc.load_gather(hn_v, [jnp.full((L,), d)])
                        return acc + c * hv
                    acc = lax.fori_loop(0, D, p1, jnp.zeros((L,), jnp.float32))

                    def p2(d, _):
                        c = plsc.load_gather(rows_v, [rows16, cols + d])
                        hv = plsc.load_gather(hn_v, [jnp.full((L,), d)])
                        tr_v[pl.ds(dslot(d, gslot), L)] = c - acc * hv
                        return 0
                    lax.fori_loop(0, D, p2, 0)
                else:
                    def p0(d, _):
                        c = plsc.load_gather(rows_v, [rows16, cols + d])
                        tr_v[pl.ds(dslot(d, gslot), L)] = c
                        return 0
                    lax.fori_loop(0, D, p0, 0)
                return 0
            lax.fori_loop(0, CHUNK // L, body, 0)
        wc = [pltpu.async_copy(
            tr_v.at[pl.ds(td * 4096, 4096)],
            outs[r].at[pl.ds(td * (BT * 1024) + base * 8, 4096)], wsem)
            for td in range(8)]
        for c in wc:
            c.wait()

    rel_outs = [r_wr, r_ci, r_co, r_af]
    for r in range(4):
        rconst = jnp.full((L,), r, jnp.int32)

        def rbody(d, _, _rc=rconst):
            bv = plsc.load_gather(rel_v, [_rc, jnp.full((L,), d)])
            def gb(gg, _):
                gslot = ((gg >> 3) << 10) + ((gg & 7) << 4)
                tr_v[pl.ds(dslot(d, gslot), L)] = bv
                return 0
            lax.fori_loop(0, CHUNK // L, gb, 0)
            return 0
        lax.fori_loop(0, D, rbody, 0)
        wc = []
        for td in range(8):
            for j in range(NCH):
                wc.append(pltpu.async_copy(
                    tr_v.at[pl.ds(td * 4096, 1024)],
                    rel_outs[r].at[pl.ds(td * (BT * 1024) + (base * 8)
                                         + j * 1024, 1024)], wsem))
        for c in wc:
            c.wait()


def kernel(user_id, wrote, cited, coauthor, affiliation,
           author_table, affil_table, doc_table, rel_table, hyper_table):
    def prep(x):
        return x.astype(jnp.int32).reshape(NW, NCH, CHUNK)

    a_pk, f_pk, d_pk = _pack_tables(author_table.T, affil_table.T,
                                    doc_table.T)

    mesh = plsc.VectorSubcoreMesh(core_axis_name="c", subcore_axis_name="s")
    g_f = pl.kernel(
        _gather_body,
        mesh=mesh,
        out_type=tuple(jax.ShapeDtypeStruct((OUT1D,), jnp.float32)
                       for _ in range(9)),
        compiler_params=pltpu.CompilerParams(use_tc_tiling_on_sc=False,
                                             needs_layout_passes=False),
        scratch_types=[
            pltpu.VMEM((NCH, CHUNK), jnp.int32),
            pltpu.VMEM((NCH, CHUNK), jnp.int32),
            pltpu.VMEM((ROWS_W,), jnp.int32),
            pltpu.VMEM((ROWS_W, 2 * D), jnp.float32),
            pltpu.VMEM((8 * NCH * 1024,), jnp.float32),
            pltpu.VMEM((D,), jnp.float32),
            pltpu.VMEM((4, D), jnp.float32),
            pltpu.VMEM((4, D), jnp.float32),
            pltpu.SemaphoreType.DMA,
            pltpu.SemaphoreType.DMA,
        ],
    )
    outs1d = g_f(prep(user_id), prep(wrote), prep(cited), prep(coauthor),
                 prep(affiliation), a_pk, f_pk, d_pk,
                 rel_table, hyper_table)

    def unbit(o):
        return (o.reshape(8, BT, 8, 128).transpose(1, 3, 0, 2)
                .reshape(B, D))

    return tuple(unbit(o) for o in outs1d)


# speedup vs baseline: 1.1138x; 1.0133x over previous
import jax
import jax.numpy as jnp
from jax import lax
from jax.experimental import pallas as pl
from jax.experimental.pallas import tpu as pltpu
from jax.experimental.pallas import tpu_sc as plsc

B = 16384
D = 64
L = 16
NC = 2
NS = 16
NW = NC * NS
ROWS_W = B // NW
CHUNK = 128
NCH = ROWS_W // CHUNK
NG = ROWS_W // L
NDR = D // L
PBLK = 1024
HALF = 49 * PBLK
BT = B // 128
OUT1D = B * D


def _lane_sum(x):
    i = lax.iota(jnp.int32, L)
    dnums = lax.GatherDimensionNumbers(
        offset_dims=(), collapsed_slice_dims=(0,), start_index_map=(0,))
    for k in (8, 4, 2, 1):
        x = x + lax.gather(x, (i ^ k)[:, None], dnums, slice_sizes=(1,),
                           mode=lax.GatherScatterMode.PROMISE_IN_BOUNDS)
    return x


def _vrsqrt(x):
    i = lax.bitcast_convert_type(x, jnp.int32)
    y = lax.bitcast_convert_type(jnp.int32(0x5F3759DF) - (i >> 1), jnp.float32)
    for _ in range(3):
        y = y * (1.5 - 0.5 * x * y * y)
    return y


def _pack_body(a0, b0, a1, b1, a2, b2, o0, o1, o2):
    ri = lax.broadcasted_iota(jnp.int32, (D, D), 0)
    ci = lax.broadcasted_iota(jnp.int32, (D, D), 1)
    ident = jnp.where(ri == ci, 1.0, 0.0).astype(jnp.float32)
    dn = (((0,), (0,)), ((), ()))
    for a, b, o in ((a0, b0, o0), (a1, b1, o1), (a2, b2, o2)):
        o[:, 0:D] = lax.dot_general(a[...], ident, dn,
                                    precision=lax.Precision.HIGHEST,
                                    preferred_element_type=jnp.float32)
        o[:, D:2 * D] = lax.dot_general(b[...], ident, dn,
                                        precision=lax.Precision.HIGHEST,
                                        preferred_element_type=jnp.float32)


def _pack_tables(a_t, f_t, d_t):
    nblk = HALF // PBLK
    in_a = pl.BlockSpec((D, PBLK), lambda t: (0, t))
    in_b = pl.BlockSpec((D, PBLK), lambda t: (0, nblk + t))
    out_s = pl.BlockSpec((PBLK, 2 * D), lambda t: (t, 0))
    return pl.pallas_call(
        _pack_body,
        grid=(nblk,),
        in_specs=[in_a, in_b, in_a, in_b, in_a, in_b],
        out_specs=[out_s, out_s, out_s],
        out_shape=tuple(jax.ShapeDtypeStruct((HALF, 2 * D), jnp.float32)
                        for _ in range(3)),
    )(a_t, a_t, f_t, f_t, d_t, d_t)


def _gather_body(uid, wr, ci, co, af, author, affil, doc, rel, hyper,
                 o_user, o_wr, o_ci, o_co, o_af, r_wr, r_ci, r_co, r_af,
                 idx_v, idx2_v, hoff_v, rows_v, tr_v, hn_v, hyp_v, rel_v,
                 gsem, wsem):
    wid = lax.axis_index("s") * NC + lax.axis_index("c")
    base = wid * ROWS_W
    pltpu.sync_copy(hyper, hyp_v)
    pltpu.sync_copy(rel, rel_v)
    lane = lax.iota(jnp.int32, L)

    idx_refs = [uid, wr, ci, co, af]
    tabs = [author, doc, doc, author, affil]
    outs = [o_user, o_wr, o_ci, o_co, o_af]

    def dslot(d, gslot):
        return ((d >> 3) << 12) + ((d & 7) << 7) + gslot

    for r in range(5):
        pltpu.sync_copy(idx_refs[r].at[wid], idx_v)
        for j in range(NCH):
            def xbody(g, _, _j=j):
                v = idx_v[_j, pl.ds(g * L, L)]
                idx2_v[_j, pl.ds(g * L, L)] = jnp.where(v >= HALF, v - HALF, v)
                hoff_v[pl.ds(_j * CHUNK + g * L, L)] = (
                    jnp.where(v >= HALF, D, 0))
                return 0
            lax.fori_loop(0, CHUNK // L, xbody, 0)
        gc = [pltpu.async_copy(tabs[r].at[idx2_v.at[j]],
                               rows_v.at[pl.ds(j * CHUNK, CHUNK)], gsem)
              for j in range(NCH)]
        if r > 0:
            h = [hyp_v[r - 1, pl.ds(dg * L, L)] for dg in range(NDR)]
            nsq = jnp.maximum(
                _lane_sum(h[0] * h[0] + h[1] * h[1] + h[2] * h[2] + h[3] * h[3]),
                1e-24)
            inv = _vrsqrt(nsq)
            for dg in range(NDR):
                hn_v[pl.ds(dg * L, L)] = h[dg] * inv
        for j in range(NCH):
            gc[j].wait()

            def body(gg, _, _j=j, _r=r):
                g = _j * (CHUNK // L) + gg
                rows16 = g * L + lane
                cols = hoff_v[pl.ds(g * L, L)]
                gslot = ((g >> 3) << 10) + ((g & 7) << 4)
                if _r > 0:
                    def pu1(u, acc):
                        cb = cols + u * 8
                        db = u * 8
                        for dd in range(8):
                            c = plsc.load_gather(rows_v, [rows16, cb + dd])
                            hv = plsc.load_gather(hn_v,
                                                  [jnp.full((L,), db + dd)])
                            acc = acc + c * hv
                        return acc
                    acc = lax.fori_loop(0, 8, pu1,
                                        jnp.zeros((L,), jnp.float32))

                    def pu2(u, _):
                        cb = cols + u * 8
                        db = u * 8
                        sb = gslot + (u << 12)
                        for dd in range(8):
                            c = plsc.load_gather(rows_v, [rows16, cb + dd])
                            hv = plsc.load_gather(hn_v,
                                                  [jnp.full((L,), db + dd)])
                            tr_v[pl.ds(sb + ((dd >> 3) << 12)
                                       + ((dd & 7) << 7), L)] = c - acc * hv
                        return 0
                    lax.fori_loop(0, 8, pu2, 0)
                else:
                    def pu0(u, _):
                        cb = cols + u * 8
                        sb = gslot + (u << 12)
                        for dd in range(8):
                            c = plsc.load_gather(rows_v, [rows16, cb + dd])
                            tr_v[pl.ds(sb + ((dd >> 3) << 12)
                                       + ((dd & 7) << 7), L)] = c
                        return 0
                    lax.fori_loop(0, 8, pu0, 0)
                return 0
            lax.fori_loop(0, CHUNK // L, body, 0)
        wc = [pltpu.async_copy(
            tr_v.at[pl.ds(td * 4096, 4096)],
            outs[r].at[pl.ds(td * (BT * 1024) + base * 8, 4096)], wsem)
            for td in range(8)]
        for c in wc:
            c.wait()

    rel_outs = [r_wr, r_ci, r_co, r_af]
    for r in range(4):
        rconst = jnp.full((L,), r, jnp.int32)

        def rbody(d, _, _rc=rconst):
            bv = plsc.load_gather(rel_v, [_rc, jnp.full((L,), d)])
            def gb(gg, _):
                gslot = ((gg >> 3) << 10) + ((gg & 7) << 4)
                tr_v[pl.ds(dslot(d, gslot), L)] = bv
                return 0
            lax.fori_loop(0, NG, gb, 0)
            return 0
        lax.fori_loop(0, D, rbody, 0)
        wc = [pltpu.async_copy(
            tr_v.at[pl.ds(td * 4096, 4096)],
            rel_outs[r].at[pl.ds(td * (BT * 1024) + base * 8, 4096)], wsem)
            for td in range(8)]
        for c in wc:
            c.wait()


def kernel(user_id, wrote, cited, coauthor, affiliation,
           author_table, affil_table, doc_table, rel_table, hyper_table):
    def prep(x):
        return x.astype(jnp.int32).reshape(NW, NCH, CHUNK)

    a_pk, f_pk, d_pk = _pack_tables(author_table.T, affil_table.T,
                                    doc_table.T)

    mesh = plsc.VectorSubcoreMesh(core_axis_name="c", subcore_axis_name="s")
    g_f = pl.kernel(
        _gather_body,
        mesh=mesh,
        out_type=tuple(jax.ShapeDtypeStruct((OUT1D,), jnp.float32)
                       for _ in range(9)),
        compiler_params=pltpu.CompilerParams(use_tc_tiling_on_sc=False,
                                             needs_layout_passes=False),
        scratch_types=[
            pltpu.VMEM((NCH, CHUNK), jnp.int32),
            pltpu.VMEM((NCH, CHUNK), jnp.int32),
            pltpu.VMEM((ROWS_W,), jnp.int32),
            pltpu.VMEM((ROWS_W, 2 * D), jnp.float32),
            pltpu.VMEM((8 * NCH * 1024,), jnp.float32),
            pltpu.VMEM((D,), jnp.float32),
            pltpu.VMEM((4, D), jnp.float32),
            pltpu.VMEM((4, D), jnp.float32),
            pltpu.SemaphoreType.DMA,
            pltpu.SemaphoreType.DMA,
        ],
    )
    outs1d = g_f(prep(user_id), prep(wrote), prep(cited), prep(coauthor),
                 prep(affiliation), a_pk, f_pk, d_pk,
                 rel_table, hyper_table)

    def unbit(o):
        return (o.reshape(8, BT, 8, 128).transpose(1, 3, 0, 2)
                .reshape(B, D))

    return tuple(unbit(o) for o in outs1d)


# speedup vs baseline: 1.6152x; 1.4502x over previous
import jax
import jax.numpy as jnp
from jax import lax
from jax.experimental import pallas as pl
from jax.experimental.pallas import tpu as pltpu
from jax.experimental.pallas import tpu_sc as plsc

B = 16384
D = 64
L = 16
NC = 2
NS = 16
NW = NC * NS
ROWS_W = B // NW
CHUNK = 128
NCH = ROWS_W // CHUNK
NG = ROWS_W // L
NDR = D // L
PBLK = 1024
HALF = 49 * PBLK
BT = B // 128
OUT1D = B * D


def _lane_sum(x):
    i = lax.iota(jnp.int32, L)
    dnums = lax.GatherDimensionNumbers(
        offset_dims=(), collapsed_slice_dims=(0,), start_index_map=(0,))
    for k in (8, 4, 2, 1):
        x = x + lax.gather(x, (i ^ k)[:, None], dnums, slice_sizes=(1,),
                           mode=lax.GatherScatterMode.PROMISE_IN_BOUNDS)
    return x


def _vrsqrt(x):
    i = lax.bitcast_convert_type(x, jnp.int32)
    y = lax.bitcast_convert_type(jnp.int32(0x5F3759DF) - (i >> 1), jnp.float32)
    for _ in range(3):
        y = y * (1.5 - 0.5 * x * y * y)
    return y


def _pack_body(a0, b0, a1, b1, a2, b2, o0, o1, o2):
    ri = lax.broadcasted_iota(jnp.int32, (D, D), 0)
    ci = lax.broadcasted_iota(jnp.int32, (D, D), 1)
    ident = jnp.where(ri == ci, 1.0, 0.0).astype(jnp.float32)
    dn = (((0,), (0,)), ((), ()))
    for a, b, o in ((a0, b0, o0), (a1, b1, o1), (a2, b2, o2)):
        o[:, 0:D] = lax.dot_general(a[...], ident, dn,
                                    precision=lax.Precision.HIGHEST,
                                    preferred_element_type=jnp.float32)
        o[:, D:2 * D] = lax.dot_general(b[...], ident, dn,
                                        precision=lax.Precision.HIGHEST,
                                        preferred_element_type=jnp.float32)


def _pack_tables(a_t, f_t, d_t):
    nblk = HALF // PBLK
    in_a = pl.BlockSpec((D, PBLK), lambda t: (0, t))
    in_b = pl.BlockSpec((D, PBLK), lambda t: (0, nblk + t))
    out_s = pl.BlockSpec((PBLK, 2 * D), lambda t: (t, 0))
    return pl.pallas_call(
        _pack_body,
        grid=(nblk,),
        in_specs=[in_a, in_b, in_a, in_b, in_a, in_b],
        out_specs=[out_s, out_s, out_s],
        out_shape=tuple(jax.ShapeDtypeStruct((HALF, 2 * D), jnp.float32)
                        for _ in range(3)),
    )(a_t, a_t, f_t, f_t, d_t, d_t)


def _gather_body(uid, wr, ci, co, af, author, affil, doc, rel, hyper,
                 o_user, o_wr, o_ci, o_co, o_af, r_wr, r_ci, r_co, r_af,
                 idx_v, idx2_v, hoff_v, stage_v, tr_v, hyp_v, rel_v,
                 gsem, wsem):
    wid = lax.axis_index("s") * NC + lax.axis_index("c")
    pltpu.sync_copy(hyper, hyp_v)
    pltpu.sync_copy(rel, rel_v)
    lane = lax.iota(jnp.int32, L)

    idx_refs = [uid, wr, ci, co, af]
    tabs = [author, doc, doc, author, affil]
    outs = [o_user, o_wr, o_ci, o_co, o_af]

    tdv = [(lane + dg * L) >> 3 for dg in range(NDR)]
    div = [(lane + dg * L) & 7 for dg in range(NDR)]

    dnums = lax.GatherDimensionNumbers(
        offset_dims=(), collapsed_slice_dims=(0,), start_index_map=(0,))

    def bcast(v, l):
        idx = jnp.full((L, 1), l, jnp.int32)
        return lax.gather(v, idx, dnums, slice_sizes=(1,),
                          mode=lax.GatherScatterMode.PROMISE_IN_BOUNDS)

    for r in range(5):
        pltpu.sync_copy(idx_refs[r].at[wid], idx_v)
        for j in range(NCH):
            def xbody(g, _, _j=j):
                v = idx_v[_j, pl.ds(g * L, L)]
                idx2_v[_j, pl.ds(g * L, L)] = jnp.where(v >= HALF, v - HALF, v)
                hoff_v[pl.ds(_j * CHUNK + g * L, L)] = (
                    jnp.where(v >= HALF, 1, 0))
                return 0
            lax.fori_loop(0, CHUNK // L, xbody, 0)
        gc = [pltpu.async_copy(tabs[r].at[idx2_v.at[j]],
                               stage_v.at[j], gsem)
              for j in range(NCH)]
        if r > 0:
            h = [hyp_v[r - 1, pl.ds(dg * L, L)] for dg in range(NDR)]
            nsq = jnp.maximum(
                _lane_sum(h[0] * h[0] + h[1] * h[1]
                          + h[2] * h[2] + h[3] * h[3]), 1e-24)
            inv = _vrsqrt(nsq)
            hn = [h[dg] * inv for dg in range(NDR)]

        for hf in range(2):
            gc[2 * hf].wait()
            gc[2 * hf + 1].wait()

            def body(gq, _, _hf=hf, _r=r):
                gj = _hf * (CHUNK // L * 2) + gq
                j = gj >> 3
                jv = jnp.full((L,), j, jnp.int32)
                hv = hoff_v[pl.ds(gj * L, L)]
                rbase = (gj & 7) << 4
                for l in range(L):
                    rowc = rbase + l
                    msk = bcast(hv, l) > 0
                    e = []
                    for dg in range(NDR):
                        lo = stage_v[j, rowc, pl.ds(dg * L, L)]
                        hi = stage_v[j, rowc, pl.ds(D + dg * L, L)]
                        e.append(jnp.where(msk, hi, lo))
                    if _r > 0:
                        p = (e[0] * hn[0] + e[1] * hn[1]
                             + e[2] * hn[2] + e[3] * hn[3])
                        s = _lane_sum(p)
                        e = [e[dg] - s * hn[dg] for dg in range(NDR)]
                    bv = jnp.full((L,), rowc, jnp.int32)
                    for dg in range(NDR):
                        plsc.store_scatter(
                            tr_v, [tdv[dg], jv, div[dg], bv], e[dg])
                return 0
            lax.fori_loop(0, CHUNK // L * 2, body, 0)
        wc = [pltpu.async_copy(
            tr_v.at[td, pl.ds(0, NCH), pl.ds(0, 8), pl.ds(0, 128)],
            outs[r].at[td, pl.ds(NCH * wid, NCH)], wsem)
            for td in range(8)]
        for c in wc:
            c.wait()

    rel_outs = [r_wr, r_ci, r_co, r_af]
    for r in range(4):
        rconst = jnp.full((L,), r, jnp.int32)

        def rbody(d, _, _rc=rconst):
            bv = plsc.load_gather(rel_v, [_rc, jnp.full((L,), d)])

            def gb(gj, _):
                tr_v[d >> 3, gj >> 3, d & 7,
                     pl.ds((gj & 7) << 4, L)] = bv
                return 0
            lax.fori_loop(0, NG, gb, 0)
            return 0
        lax.fori_loop(0, D, rbody, 0)
        wc = [pltpu.async_copy(
            tr_v.at[td, pl.ds(0, NCH), pl.ds(0, 8), pl.ds(0, 128)],
            rel_outs[r].at[td, pl.ds(NCH * wid, NCH)], wsem)
            for td in range(8)]
        for c in wc:
            c.wait()


def kernel(user_id, wrote, cited, coauthor, affiliation,
           author_table, affil_table, doc_table, rel_table, hyper_table):
    def prep(x):
        return x.astype(jnp.int32).reshape(NW, NCH, CHUNK)

    a_pk, f_pk, d_pk = _pack_tables(author_table.T, affil_table.T,
                                    doc_table.T)

    mesh = plsc.VectorSubcoreMesh(core_axis_name="c", subcore_axis_name="s")
    g_f = pl.kernel(
        _gather_body,
        mesh=mesh,
        out_type=tuple(jax.ShapeDtypeStruct((8, BT, 8, 128), jnp.float32)
                       for _ in range(9)),
        compiler_params=pltpu.CompilerParams(use_tc_tiling_on_sc=False,
                                             needs_layout_passes=False),
        scratch_types=[
            pltpu.VMEM((NCH, CHUNK), jnp.int32),
            pltpu.VMEM((NCH, CHUNK), jnp.int32),
            pltpu.VMEM((ROWS_W,), jnp.int32),
            pltpu.VMEM((NCH, CHUNK, 2 * D), jnp.float32),
            pltpu.VMEM((8, NCH, 8, 129), jnp.float32),
            pltpu.VMEM((4, D), jnp.float32),
            pltpu.VMEM((4, D), jnp.float32),
            pltpu.SemaphoreType.DMA,
            pltpu.SemaphoreType.DMA,
        ],
    )
    outs4 = g_f(prep(user_id), prep(wrote), prep(cited), prep(coauthor),
                prep(affiliation), a_pk, f_pk, d_pk,
                rel_table, hyper_table)

    def unbit(o):
        return o.transpose(1, 3, 0, 2).reshape(B, D)

    return tuple(unbit(o) for o in outs4)


# speedup vs baseline: 2.0300x; 1.2568x over previous
import jax
import jax.numpy as jnp
from jax import lax
from jax.experimental import pallas as pl
from jax.experimental.pallas import tpu as pltpu
from jax.experimental.pallas import tpu_sc as plsc

B = 16384
D = 64
L = 16
NC = 2
NS = 16
NW = NC * NS
ROWS_W = B // NW
CHUNK = 128
NCH = ROWS_W // CHUNK
NG = ROWS_W // L
NDR = D // L
PBLK = 1024
HALF = 49 * PBLK
BT = B // 128
OUT1D = B * D


def _lane_sum(x):
    i = lax.iota(jnp.int32, L)
    dnums = lax.GatherDimensionNumbers(
        offset_dims=(), collapsed_slice_dims=(0,), start_index_map=(0,))
    for k in (8, 4, 2, 1):
        x = x + lax.gather(x, (i ^ k)[:, None], dnums, slice_sizes=(1,),
                           mode=lax.GatherScatterMode.PROMISE_IN_BOUNDS)
    return x


def _vrsqrt(x):
    i = lax.bitcast_convert_type(x, jnp.int32)
    y = lax.bitcast_convert_type(jnp.int32(0x5F3759DF) - (i >> 1), jnp.float32)
    for _ in range(3):
        y = y * (1.5 - 0.5 * x * y * y)
    return y


def _pack_body(a0, b0, a1, b1, a2, b2, o0, o1, o2):
    ri = lax.broadcasted_iota(jnp.int32, (D, D), 0)
    ci = lax.broadcasted_iota(jnp.int32, (D, D), 1)
    ident = jnp.where(ri == ci, 1.0, 0.0).astype(jnp.float32)
    dn = (((0,), (0,)), ((), ()))
    for a, b, o in ((a0, b0, o0), (a1, b1, o1), (a2, b2, o2)):
        o[:, 0:D] = lax.dot_general(a[...], ident, dn,
                                    preferred_element_type=jnp.float32)
        o[:, D:2 * D] = lax.dot_general(b[...], ident, dn,
                                        preferred_element_type=jnp.float32)


def _pack_tables(a_t, f_t, d_t):
    nblk = HALF // PBLK
    in_a = pl.BlockSpec((D, PBLK), lambda t: (0, t))
    in_b = pl.BlockSpec((D, PBLK), lambda t: (0, nblk + t))
    out_s = pl.BlockSpec((PBLK, 2 * D), lambda t: (t, 0))
    return pl.pallas_call(
        _pack_body,
        grid=(nblk,),
        in_specs=[in_a, in_b, in_a, in_b, in_a, in_b],
        out_specs=[out_s, out_s, out_s],
        out_shape=tuple(jax.ShapeDtypeStruct((HALF, 2 * D), jnp.float32)
                        for _ in range(3)),
    )(a_t, a_t, f_t, f_t, d_t, d_t)


def _gather_body(uid, wr, ci, co, af, author, affil, doc, rel, hyper,
                 o_user, o_wr, o_ci, o_co, o_af, r_wr, r_ci, r_co, r_af,
                 idx_v, idx2_v, hoff_v, stage_v, tr_v, hyp_v, rel_v,
                 gsem, wsem):
    wid = lax.axis_index("s") * NC + lax.axis_index("c")
    pltpu.sync_copy(hyper, hyp_v)
    pltpu.sync_copy(rel, rel_v)
    lane = lax.iota(jnp.int32, L)

    idx_refs = [uid, wr, ci, co, af]
    tabs = [author, doc, doc, author, affil]
    outs = [o_user, o_wr, o_ci, o_co, o_af]

    tdv = [(lane + dg * L) >> 3 for dg in range(NDR)]
    div = [(lane + dg * L) & 7 for dg in range(NDR)]

    dnums = lax.GatherDimensionNumbers(
        offset_dims=(), collapsed_slice_dims=(0,), start_index_map=(0,))

    def bcast(v, l):
        idx = jnp.full((L, 1), l, jnp.int32)
        return lax.gather(v, idx, dnums, slice_sizes=(1,),
                          mode=lax.GatherScatterMode.PROMISE_IN_BOUNDS)

    for r in range(5):
        pltpu.sync_copy(idx_refs[r].at[wid], idx_v)
        for j in range(NCH):
            def xbody(g, _, _j=j):
                v = idx_v[_j, pl.ds(g * L, L)]
                idx2_v[_j, pl.ds(g * L, L)] = jnp.where(v >= HALF, v - HALF, v)
                hoff_v[pl.ds(_j * CHUNK + g * L, L)] = (
                    jnp.where(v >= HALF, 1, 0))
                return 0
            lax.fori_loop(0, CHUNK // L, xbody, 0)
        gc = [pltpu.async_copy(tabs[r].at[idx2_v.at[j]],
                               stage_v.at[j], gsem)
              for j in range(NCH)]
        if r > 0:
            h = [hyp_v[r - 1, pl.ds(dg * L, L)] for dg in range(NDR)]
            nsq = jnp.maximum(
                _lane_sum(h[0] * h[0] + h[1] * h[1]
                          + h[2] * h[2] + h[3] * h[3]), 1e-24)
            inv = _vrsqrt(nsq)
            hn = [h[dg] * inv for dg in range(NDR)]

        for hf in range(2):
            gc[2 * hf].wait()
            gc[2 * hf + 1].wait()

            def body(gq, _, _hf=hf, _r=r):
                gj = _hf * (CHUNK // L * 2) + gq
                j = gj >> 3
                jv = jnp.full((L,), j, jnp.int32)
                hv = hoff_v[pl.ds(gj * L, L)]
                rbase = (gj & 7) << 4
                for l in range(L):
                    rowc = rbase + l
                    msk = bcast(hv, l) > 0
                    e = []
                    for dg in range(NDR):
                        lo = stage_v[j, rowc, pl.ds(dg * L, L)]
                        hi = stage_v[j, rowc, pl.ds(D + dg * L, L)]
                        e.append(jnp.where(msk, hi, lo))
                    if _r > 0:
                        p = (e[0] * hn[0] + e[1] * hn[1]
                             + e[2] * hn[2] + e[3] * hn[3])
                        s = _lane_sum(p)
                        e = [e[dg] - s * hn[dg] for dg in range(NDR)]
                    bv = jnp.full((L,), rowc, jnp.int32)
                    for dg in range(NDR):
                        plsc.store_scatter(
                            tr_v, [tdv[dg], jv, div[dg], bv], e[dg])
                return 0
            lax.fori_loop(0, CHUNK // L * 2, body, 0)
        wc = [pltpu.async_copy(
            tr_v.at[td, pl.ds(0, NCH), pl.ds(0, 8), pl.ds(0, 128)],
            outs[r].at[td, pl.ds(NCH * wid, NCH)], wsem)
            for td in range(8)]
        for c in wc:
            c.wait()

    rel_outs = [r_wr, r_ci, r_co, r_af]
    for r in range(4):
        rconst = jnp.full((L,), r, jnp.int32)

        def rbody(d, _, _rc=rconst):
            bv = plsc.load_gather(rel_v, [_rc, jnp.full((L,), d)])

            def gb(gj, _):
                tr_v[d >> 3, gj >> 3, d & 7,
                     pl.ds((gj & 7) << 4, L)] = bv
                return 0
            lax.fori_loop(0, NG, gb, 0)
            return 0
        lax.fori_loop(0, D, rbody, 0)
        wc = [pltpu.async_copy(
            tr_v.at[td, pl.ds(0, NCH), pl.ds(0, 8), pl.ds(0, 128)],
            rel_outs[r].at[td, pl.ds(NCH * wid, NCH)], wsem)
            for td in range(8)]
        for c in wc:
            c.wait()


def kernel(user_id, wrote, cited, coauthor, affiliation,
           author_table, affil_table, doc_table, rel_table, hyper_table):
    def prep(x):
        return x.astype(jnp.int32).reshape(NW, NCH, CHUNK)

    a_pk, f_pk, d_pk = _pack_tables(author_table.T, affil_table.T,
                                    doc_table.T)

    mesh = plsc.VectorSubcoreMesh(core_axis_name="c", subcore_axis_name="s")
    g_f = pl.kernel(
        _gather_body,
        mesh=mesh,
        out_type=tuple(jax.ShapeDtypeStruct((8, BT, 8, 128), jnp.float32)
                       for _ in range(9)),
        compiler_params=pltpu.CompilerParams(use_tc_tiling_on_sc=False,
                                             needs_layout_passes=False),
        scratch_types=[
            pltpu.VMEM((NCH, CHUNK), jnp.int32),
            pltpu.VMEM((NCH, CHUNK), jnp.int32),
            pltpu.VMEM((ROWS_W,), jnp.int32),
            pltpu.VMEM((NCH, CHUNK, 2 * D), jnp.float32),
            pltpu.VMEM((8, NCH, 8, 129), jnp.float32),
            pltpu.VMEM((4, D), jnp.float32),
            pltpu.VMEM((4, D), jnp.float32),
            pltpu.SemaphoreType.DMA,
            pltpu.SemaphoreType.DMA,
        ],
    )
    outs4 = g_f(prep(user_id), prep(wrote), prep(cited), prep(coauthor),
                prep(affiliation), a_pk, f_pk, d_pk,
                rel_table, hyper_table)

    def unbit(o):
        return o.transpose(1, 3, 0, 2).reshape(B, D)

    return tuple(unbit(o) for o in outs4)


# speedup vs baseline: 2.0971x; 1.0331x over previous
import jax
import jax.numpy as jnp
from jax import lax
from jax.experimental import pallas as pl
from jax.experimental.pallas import tpu as pltpu
from jax.experimental.pallas import tpu_sc as plsc

B = 16384
D = 64
L = 16
NC = 2
NS = 16
NW = NC * NS
ROWS_W = B // NW
CHUNK = 128
NCH = ROWS_W // CHUNK
NG = ROWS_W // L
NDR = D // L
PBLK = 1024
HALF = 49 * PBLK
BT = B // 128
OUT1D = B * D


def _lane_sum(x):
    i = lax.iota(jnp.int32, L)
    dnums = lax.GatherDimensionNumbers(
        offset_dims=(), collapsed_slice_dims=(0,), start_index_map=(0,))
    for k in (8, 4, 2, 1):
        x = x + lax.gather(x, (i ^ k)[:, None], dnums, slice_sizes=(1,),
                           mode=lax.GatherScatterMode.PROMISE_IN_BOUNDS)
    return x


def _vrsqrt(x):
    i = lax.bitcast_convert_type(x, jnp.int32)
    y = lax.bitcast_convert_type(jnp.int32(0x5F3759DF) - (i >> 1), jnp.float32)
    for _ in range(3):
        y = y * (1.5 - 0.5 * x * y * y)
    return y


def _pack_body(a, b, o):
    ri = lax.broadcasted_iota(jnp.int32, (D, D), 0)
    ci = lax.broadcasted_iota(jnp.int32, (D, D), 1)
    ident = jnp.where(ri == ci, 1.0, 0.0).astype(jnp.float32)
    dn = (((0,), (0,)), ((), ()))
    o[:, 0:D] = lax.dot_general(a[...], ident, dn,
                                preferred_element_type=jnp.float32)
    o[:, D:2 * D] = lax.dot_general(b[...], ident, dn,
                                    preferred_element_type=jnp.float32)


def _pack_table(t):
    nblk = HALF // PBLK
    in_a = pl.BlockSpec((D, PBLK), lambda t: (0, t))
    in_b = pl.BlockSpec((D, PBLK), lambda t: (0, nblk + t))
    out_s = pl.BlockSpec((PBLK, 2 * D), lambda t: (t, 0))
    return pl.pallas_call(
        _pack_body,
        grid=(nblk,),
        in_specs=[in_a, in_b],
        out_specs=out_s,
        out_shape=jax.ShapeDtypeStruct((HALF, 2 * D), jnp.float32),
    )(t, t)


def _make_sc_body(prs, nbc):
    n = len(prs)

    def body(*args):
        p = 0
        idx_refs = args[p:p + n]; p += n
        tab = args[p]; p += 1
        hyp = args[p]; p += 1
        rel = args[p] if nbc else None
        p += 1 if nbc else 0
        outs = args[p:p + n]; p += n
        rel_outs = args[p:p + 4] if nbc else ()
        p += 4 if nbc else 0
        (idx_v, idx2_v, hoff_v, stage_v, tr_v, hyp_v, rel_v,
         gsem, wsem) = args[p:]

        wid = lax.axis_index("s") * NC + lax.axis_index("c")
        pltpu.sync_copy(hyp, hyp_v)
        if nbc:
            pltpu.sync_copy(rel, rel_v)
        lane = lax.iota(jnp.int32, L)

        tdv = [(lane + dg * L) >> 3 for dg in range(NDR)]
        div = [(lane + dg * L) & 7 for dg in range(NDR)]

        dnums = lax.GatherDimensionNumbers(
            offset_dims=(), collapsed_slice_dims=(0,), start_index_map=(0,))

        def bcast(v, l):
            idx = jnp.full((L, 1), l, jnp.int32)
            return lax.gather(v, idx, dnums, slice_sizes=(1,),
                              mode=lax.GatherScatterMode.PROMISE_IN_BOUNDS)

        for r in range(n):
            pltpu.sync_copy(idx_refs[r].at[wid], idx_v)
            for j in range(NCH):
                def xbody(g, _, _j=j):
                    v = idx_v[_j, pl.ds(g * L, L)]
                    idx2_v[_j, pl.ds(g * L, L)] = (
                        jnp.where(v >= HALF, v - HALF, v))
                    hoff_v[pl.ds(_j * CHUNK + g * L, L)] = (
                        jnp.where(v >= HALF, 1, 0))
                    return 0
                lax.fori_loop(0, CHUNK // L, xbody, 0)
            gc = [pltpu.async_copy(tab.at[idx2_v.at[j]],
                                   stage_v.at[j], gsem)
                  for j in range(NCH)]
            hrow = prs[r]
            if hrow is not None:
                h = [hyp_v[hrow, pl.ds(dg * L, L)] for dg in range(NDR)]
                nsq = jnp.maximum(
                    _lane_sum(h[0] * h[0] + h[1] * h[1]
                              + h[2] * h[2] + h[3] * h[3]), 1e-24)
                inv = _vrsqrt(nsq)
                hn = [h[dg] * inv for dg in range(NDR)]

            for hf in range(2):
                gc[2 * hf].wait()
                gc[2 * hf + 1].wait()

                def body_g(gq, _, _hf=hf, _hr=hrow):
                    gj = _hf * (CHUNK // L * 2) + gq
                    j = gj >> 3
                    jv = jnp.full((L,), j, jnp.int32)
                    hv = hoff_v[pl.ds(gj * L, L)]
                    rbase = (gj & 7) << 4
                    for l in range(L):
                        rowc = rbase + l
                        msk = bcast(hv, l) > 0
                        e = []
                        for dg in range(NDR):
                            lo = stage_v[j, rowc, pl.ds(dg * L, L)]
                            hi = stage_v[j, rowc, pl.ds(D + dg * L, L)]
                            e.append(jnp.where(msk, hi, lo))
                        if _hr is not None:
                            pp = (e[0] * hn[0] + e[1] * hn[1]
                                  + e[2] * hn[2] + e[3] * hn[3])
                            s = _lane_sum(pp)
                            e = [e[dg] - s * hn[dg] for dg in range(NDR)]
                        bv = jnp.full((L,), rowc, jnp.int32)
                        for dg in range(NDR):
                            plsc.store_scatter(
                                tr_v, [tdv[dg], jv, div[dg], bv], e[dg])
                    return 0
                lax.fori_loop(0, CHUNK // L * 2, body_g, 0)
            wc = [pltpu.async_copy(
                tr_v.at[td, pl.ds(0, NCH), pl.ds(0, 8), pl.ds(0, 128)],
                outs[r].at[td, pl.ds(NCH * wid, NCH)], wsem)
                for td in range(8)]
            for c in wc:
                c.wait()

        for r in range(4 if nbc else 0):
            rconst = jnp.full((L,), r, jnp.int32)

            def rbody(d, _, _rc=rconst):
                bvv = plsc.load_gather(rel_v, [_rc, jnp.full((L,), d)])

                def gb(gj, _):
                    tr_v[d >> 3, gj >> 3, d & 7,
                         pl.ds((gj & 7) << 4, L)] = bvv
                    return 0
                lax.fori_loop(0, NG, gb, 0)
                return 0
            lax.fori_loop(0, D, rbody, 0)
            wc = [pltpu.async_copy(
                tr_v.at[td, pl.ds(0, NCH), pl.ds(0, 8), pl.ds(0, 128)],
                rel_outs[r].at[td, pl.ds(NCH * wid, NCH)], wsem)
                for td in range(8)]
            for c in wc:
                c.wait()

    return body


_MESH = plsc.VectorSubcoreMesh(core_axis_name="c", subcore_axis_name="s")
_SC_PARAMS = pltpu.CompilerParams(use_tc_tiling_on_sc=False,
                                  needs_layout_passes=False)


def _sc_call(prs, nbc, n_out):
    return pl.kernel(
        _make_sc_body(prs, nbc),
        mesh=_MESH,
        out_type=tuple(jax.ShapeDtypeStruct((8, BT, 8, 128), jnp.float32)
                       for _ in range(n_out)),
        compiler_params=_SC_PARAMS,
        scratch_types=[
            pltpu.VMEM((NCH, CHUNK), jnp.int32),
            pltpu.VMEM((NCH, CHUNK), jnp.int32),
            pltpu.VMEM((ROWS_W,), jnp.int32),
            pltpu.VMEM((NCH, CHUNK, 2 * D), jnp.float32),
            pltpu.VMEM((8, NCH, 8, 129), jnp.float32),
            pltpu.VMEM((4, D), jnp.float32),
            pltpu.VMEM((4, D), jnp.float32),
            pltpu.SemaphoreType.DMA,
            pltpu.SemaphoreType.DMA,
        ],
    )


def kernel(user_id, wrote, cited, coauthor, affiliation,
           author_table, affil_table, doc_table, rel_table, hyper_table):
    def prep(x):
        return x.astype(jnp.int32).reshape(NW, NCH, CHUNK)

    a_pk = _pack_table(author_table.T)
    f_a = _sc_call([None, 2], False, 2)
    o_user, o_co = f_a(prep(user_id), prep(coauthor), a_pk, hyper_table)

    d_pk = _pack_table(doc_table.T)
    f_d = _sc_call([0, 1], False, 2)
    o_wr, o_ci = f_d(prep(wrote), prep(cited), d_pk, hyper_table)

    f_pk = _pack_table(affil_table.T)
    f_f = _sc_call([3], True, 5)
    o_af, r_wr, r_ci, r_co, r_af = f_f(prep(affiliation), f_pk, hyper_table,
                                       rel_table)

    def unbit(o):
        return o.transpose(1, 3, 0, 2).reshape(B, D)

    return tuple(unbit(o) for o in
                 (o_user, o_wr, o_ci, o_co, o_af, r_wr, r_ci, r_co, r_af))


# speedup vs baseline: 2.3122x; 1.1026x over previous
import jax
import jax.numpy as jnp
from jax import lax
from jax.experimental import pallas as pl
from jax.experimental.pallas import tpu as pltpu
from jax.experimental.pallas import tpu_sc as plsc

B = 16384
D = 64
L = 16
NC = 2
NS = 16
NW = NC * NS
ROWS_W = B // NW
CHUNK = 128
NCH = ROWS_W // CHUNK
NG = ROWS_W // L
NDR = D // L
PBLK = 1024
HALF = 49 * PBLK
BT = B // 128
OUT1D = B * D


def _lane_sum(x):
    i = lax.iota(jnp.int32, L)
    dnums = lax.GatherDimensionNumbers(
        offset_dims=(), collapsed_slice_dims=(0,), start_index_map=(0,))
    for k in (8, 4, 2, 1):
        x = x + lax.gather(x, (i ^ k)[:, None], dnums, slice_sizes=(1,),
                           mode=lax.GatherScatterMode.PROMISE_IN_BOUNDS)
    return x


def _vrsqrt(x):
    i = lax.bitcast_convert_type(x, jnp.int32)
    y = lax.bitcast_convert_type(jnp.int32(0x5F3759DF) - (i >> 1), jnp.float32)
    for _ in range(3):
        y = y * (1.5 - 0.5 * x * y * y)
    return y


def _pack_body(a, b, o):
    ri = lax.broadcasted_iota(jnp.int32, (D, D), 0)
    ci = lax.broadcasted_iota(jnp.int32, (D, D), 1)
    ident = jnp.where(ri == ci, 1.0, 0.0).astype(jnp.float32)
    dn = (((0,), (0,)), ((), ()))
    o[:, 0:D] = lax.dot_general(a[...], ident, dn,
                                preferred_element_type=jnp.float32)
    o[:, D:2 * D] = lax.dot_general(b[...], ident, dn,
                                    preferred_element_type=jnp.float32)


def _pack_table(t):
    nblk = HALF // PBLK
    in_a = pl.BlockSpec((D, PBLK), lambda t: (0, t))
    in_b = pl.BlockSpec((D, PBLK), lambda t: (0, nblk + t))
    out_s = pl.BlockSpec((PBLK, 2 * D), lambda t: (t, 0))
    return pl.pallas_call(
        _pack_body,
        grid=(nblk,),
        in_specs=[in_a, in_b],
        out_specs=out_s,
        out_shape=jax.ShapeDtypeStruct((HALF, 2 * D), jnp.float32),
    )(t, t)


def _make_sc_body(prs, nbc):
    n = len(prs)

    def body(*args):
        p = 0
        idx_refs = args[p:p + n]; p += n
        tab = args[p]; p += 1
        hyp = args[p]; p += 1
        rel = args[p] if nbc else None
        p += 1 if nbc else 0
        outs = args[p:p + n]; p += n
        rel_outs = args[p:p + 4] if nbc else ()
        p += 4 if nbc else 0
        (idx_v, idx2_v, hoff_v, stage_v, tr_v, hyp_v, rel_v,
         gsem, wsem) = args[p:]

        wid = lax.axis_index("s") * NC + lax.axis_index("c")
        pltpu.sync_copy(hyp, hyp_v)
        if nbc:
            pltpu.sync_copy(rel, rel_v)
        lane = lax.iota(jnp.int32, L)

        tdv = [(lane + dg * L) >> 3 for dg in range(NDR)]
        div = [(lane + dg * L) & 7 for dg in range(NDR)]

        dnums = lax.GatherDimensionNumbers(
            offset_dims=(), collapsed_slice_dims=(0,), start_index_map=(0,))

        def bcast(v, l):
            idx = jnp.full((L, 1), l, jnp.int32)
            return lax.gather(v, idx, dnums, slice_sizes=(1,),
                              mode=lax.GatherScatterMode.PROMISE_IN_BOUNDS)

        for r in range(n):
            pltpu.sync_copy(idx_refs[r].at[wid], idx_v)
            for j in range(NCH):
                def xbody(g, _, _j=j):
                    v = idx_v[_j, pl.ds(g * L, L)]
                    idx2_v[_j, pl.ds(g * L, L)] = (
                        jnp.where(v >= HALF, v - HALF, v))
                    hoff_v[pl.ds(_j * CHUNK + g * L, L)] = (
                        jnp.where(v >= HALF, 1, 0))
                    return 0
                lax.fori_loop(0, CHUNK // L, xbody, 0)
            gc = [pltpu.async_copy(tab.at[idx2_v.at[j]],
                                   stage_v.at[j], gsem)
                  for j in range(NCH)]
            hrow = prs[r]
            if hrow is not None:
                h = [hyp_v[hrow, pl.ds(dg * L, L)] for dg in range(NDR)]
                nsq = jnp.maximum(
                    _lane_sum(h[0] * h[0] + h[1] * h[1]
                              + h[2] * h[2] + h[3] * h[3]), 1e-24)
                inv = _vrsqrt(nsq)
                hn = [h[dg] * inv for dg in range(NDR)]

            for hf in range(2):
                gc[2 * hf].wait()
                gc[2 * hf + 1].wait()

                def body_g(gq, _, _hf=hf, _hr=hrow):
                    gj = _hf * (CHUNK // L * 2) + gq
                    j = gj >> 3
                    jv = jnp.full((L,), j, jnp.int32)
                    hv = hoff_v[pl.ds(gj * L, L)]
                    rbase = (gj & 7) << 4
                    for l in range(L):
                        rowc = rbase + l
                        msk = bcast(hv, l) > 0
                        e = []
                        for dg in range(NDR):
                            lo = stage_v[j, rowc, pl.ds(dg * L, L)]
                            hi = stage_v[j, rowc, pl.ds(D + dg * L, L)]
                            e.append(jnp.where(msk, hi, lo))
                        if _hr is not None:
                            pp = (e[0] * hn[0] + e[1] * hn[1]
                                  + e[2] * hn[2] + e[3] * hn[3])
                            s = _lane_sum(pp)
                            e = [e[dg] - s * hn[dg] for dg in range(NDR)]
                        bv = jnp.full((L,), rowc, jnp.int32)
                        for dg in range(NDR):
                            plsc.store_scatter(
                                tr_v, [tdv[dg], jv, div[dg], bv], e[dg])
                    return 0
                lax.fori_loop(0, CHUNK // L * 2, body_g, 0)
            wc = [pltpu.async_copy(
                tr_v.at[td, pl.ds(0, NCH), pl.ds(0, 8), pl.ds(0, 128)],
                outs[r].at[td, pl.ds(NCH * wid, NCH)], wsem)
                for td in range(8)]
            for c in wc:
                c.wait()

        for r in range(4 if nbc else 0):
            rconst = jnp.full((L,), r, jnp.int32)

            def rbody(d, _, _rc=rconst):
                bvv = plsc.load_gather(rel_v, [_rc, jnp.full((L,), d)])
                td = d >> 3
                di = d & 7

                def gb(jj, _):
                    for q in range(8):
                        tr_v[td, jj, di, pl.ds(q << 4, L)] = bvv
                    return 0
                lax.fori_loop(0, NCH, gb, 0)
                return 0
            lax.fori_loop(0, D, rbody, 0)
            wc = [pltpu.async_copy(
                tr_v.at[td, pl.ds(0, NCH), pl.ds(0, 8), pl.ds(0, 128)],
                rel_outs[r].at[td, pl.ds(NCH * wid, NCH)], wsem)
                for td in range(8)]
            for c in wc:
                c.wait()

    return body


_MESH = plsc.VectorSubcoreMesh(core_axis_name="c", subcore_axis_name="s")
_SC_PARAMS = pltpu.CompilerParams(use_tc_tiling_on_sc=False,
                                  needs_layout_passes=False)


def _sc_call(prs, nbc, n_out):
    return pl.kernel(
        _make_sc_body(prs, nbc),
        mesh=_MESH,
        out_type=tuple(jax.ShapeDtypeStruct((8, BT, 8, 128), jnp.float32)
                       for _ in range(n_out)),
        compiler_params=_SC_PARAMS,
        scratch_types=[
            pltpu.VMEM((NCH, CHUNK), jnp.int32),
            pltpu.VMEM((NCH, CHUNK), jnp.int32),
            pltpu.VMEM((ROWS_W,), jnp.int32),
            pltpu.VMEM((NCH, CHUNK, 2 * D), jnp.float32),
            pltpu.VMEM((8, NCH, 8, 129), jnp.float32),
            pltpu.VMEM((4, D), jnp.float32),
            pltpu.VMEM((4, D), jnp.float32),
            pltpu.SemaphoreType.DMA,
            pltpu.SemaphoreType.DMA,
        ],
    )


def kernel(user_id, wrote, cited, coauthor, affiliation,
           author_table, affil_table, doc_table, rel_table, hyper_table):
    def prep(x):
        return x.astype(jnp.int32).reshape(NW, NCH, CHUNK)

    a_pk = _pack_table(author_table.T)
    f_a = _sc_call([None, 2], False, 2)
    o_user, o_co = f_a(prep(user_id), prep(coauthor), a_pk, hyper_table)

    d_pk = _pack_table(doc_table.T)
    f_d = _sc_call([0, 1], False, 2)
    o_wr, o_ci = f_d(prep(wrote), prep(cited), d_pk, hyper_table)

    f_pk = _pack_table(affil_table.T)
    f_f = _sc_call([3], True, 5)
    o_af, r_wr, r_ci, r_co, r_af = f_f(prep(affiliation), f_pk, hyper_table,
                                       rel_table)

    def unbit(o):
        return o.transpose(1, 3, 0, 2).reshape(B, D)

    return tuple(unbit(o) for o in
                 (o_user, o_wr, o_ci, o_co, o_af, r_wr, r_ci, r_co, r_af))
